# baseline (speedup 1.0000x reference)
import functools

import jax
import jax.numpy as jnp
from jax import lax
from jax.experimental import pallas as pl
from jax.experimental.pallas import tpu as pltpu
from jax.experimental.pallas import tpu_sc as plsc

MAX_RES_OFFSET = 32
MAX_SYM_OFFSET = 2
D = 32
N = 1024
R_ROWS = 2 * MAX_RES_OFFSET + 2
S_ROWS = 2 * MAX_SYM_OFFSET + 2
C_ROWS = R_ROWS * S_ROWS

NUM_CORES = 2
NUM_SUBCORES = 16
NUM_WORKERS = NUM_CORES * NUM_SUBCORES
ROWS_PER_WORKER = N // NUM_WORKERS
L = 16
PAD = 33

_SC_PARAMS = dict(
    compiler_params=pltpu.CompilerParams(
        needs_layout_passes=False, use_tc_tiling_on_sc=False
    ),
)


def _worker_id():
    return lax.axis_index("s") * NUM_CORES + lax.axis_index("c")


@functools.cache
def _make_kernels():
    mesh = plsc.VectorSubcoreMesh(core_axis_name="c", subcore_axis_name="s")

    @functools.partial(
        pl.kernel,
        out_type=jax.ShapeDtypeStruct((C_ROWS * D,), jnp.float32),
        mesh=mesh,
        scratch_types=[
            pltpu.VMEM((R_ROWS, D), jnp.float32),
            pltpu.VMEM((S_ROWS, D), jnp.float32),
            pltpu.VMEM((C_ROWS * D,), jnp.float32),
        ],
        **_SC_PARAMS,
    )
    def build_comb(res_hbm, sym_hbm, comb_hbm, res_v, sym_v, comb_v):
        wid = _worker_id()

        @pl.when(wid == 0)
        def _():
            pltpu.sync_copy(res_hbm, res_v)
            pltpu.sync_copy(sym_hbm, sym_v)
            sym_rows = [
                [sym_v[b, pl.ds(h * L, L)] for h in range(D // L)]
                for b in range(S_ROWS)
            ]

            def body(a, carry):
                for h in range(D // L):
                    r = res_v[a, pl.ds(h * L, L)]
                    for b in range(S_ROWS):
                        comb_v[pl.ds((a * S_ROWS + b) * D + h * L, L)] = (
                            r + sym_rows[b][h]
                        )
                return carry

            lax.fori_loop(0, R_ROWS, body, 0)
            pltpu.sync_copy(comb_v, comb_hbm)

    @functools.partial(
        pl.kernel,
        out_type=jax.ShapeDtypeStruct((N, N, D), jnp.float32),
        mesh=mesh,
        scratch_types=[
            pltpu.VMEM((N,), jnp.int32),
            pltpu.VMEM((N,), jnp.int32),
            pltpu.VMEM((N,), jnp.int32),
            pltpu.VMEM((N,), jnp.int32),
            pltpu.VMEM((ROWS_PER_WORKER,), jnp.int32),
            pltpu.VMEM((ROWS_PER_WORKER,), jnp.int32),
            pltpu.VMEM((ROWS_PER_WORKER,), jnp.int32),
            pltpu.VMEM((ROWS_PER_WORKER,), jnp.int32),
            pltpu.VMEM((C_ROWS * D,), jnp.float32),
            pltpu.VMEM((C_ROWS * PAD,), jnp.float32),
            pltpu.VMEM((N, PAD), jnp.float32),
            pltpu.VMEM((N, PAD), jnp.float32),
            pltpu.SemaphoreType.DMA,
            pltpu.SemaphoreType.DMA,
        ],
        **_SC_PARAMS,
    )
    def edge_embed(
        ri_hbm, ch_hbm, en_hbm, sy_hbm, comb_hbm, out_hbm,
        ri_v, ch_v, en_v, sy_v, qri_v, qch_v, qen_v, qsy_v, comb_v, comb_p,
        rows_a, rows_b, sem_a, sem_b,
    ):
        wid = _worker_id()
        base = wid * ROWS_PER_WORKER

        pltpu.sync_copy(ri_hbm, ri_v)
        pltpu.sync_copy(ch_hbm, ch_v)
        pltpu.sync_copy(en_hbm, en_v)
        pltpu.sync_copy(sy_hbm, sy_v)
        pltpu.sync_copy(ri_hbm.at[pl.ds(base, ROWS_PER_WORKER)], qri_v)
        pltpu.sync_copy(ch_hbm.at[pl.ds(base, ROWS_PER_WORKER)], qch_v)
        pltpu.sync_copy(en_hbm.at[pl.ds(base, ROWS_PER_WORKER)], qen_v)
        pltpu.sync_copy(sy_hbm.at[pl.ds(base, ROWS_PER_WORKER)], qsy_v)
        pltpu.sync_copy(comb_hbm, comb_v)

        iota = lax.iota(jnp.int32, L)

        def pad_body(r, carry):
            for h in range(D // L):
                v = comb_v[pl.ds(r * D + h * L, L)]
                plsc.store_scatter(comb_p, [iota + (r * PAD + h * L)], v)
            return carry

        lax.fori_loop(0, C_ROWS, pad_body, 0)

        def fill_row(il, slab):
            splat_il = jnp.full((L,), il, jnp.int32)
            q_ri = plsc.load_gather(qri_v, [splat_il])
            q_ch = plsc.load_gather(qch_v, [splat_il])
            q_en = plsc.load_gather(qen_v, [splat_il])
            q_sy = plsc.load_gather(qsy_v, [splat_il])

            @plsc.parallel_loop(0, N // L, 1, unroll=2)
            def grp_body(g):
                j0 = g * L
                rj = ri_v[pl.ds(j0, L)]
                cj = ch_v[pl.ds(j0, L)]
                ej = en_v[pl.ds(j0, L)]
                sj = sy_v[pl.ds(j0, L)]
                rel = jnp.clip(rj - q_ri + MAX_RES_OFFSET, 0, 2 * MAX_RES_OFFSET)
                rel = jnp.where(cj == q_ch, rel, 2 * MAX_RES_OFFSET + 1)
                rs = jnp.clip(sj - q_sy + MAX_SYM_OFFSET, 0, 2 * MAX_SYM_OFFSET)
                rs = jnp.where(ej == q_en, rs, 2 * MAX_SYM_OFFSET + 1)
                src0 = (rel * S_ROWS + rs) * PAD
                jrow = iota + j0
                for d0 in range(0, D, 8):
                    vals = [
                        plsc.load_gather(comb_p, [src0 + d])
                        for d in range(d0, d0 + 8)
                    ]
                    for i, d in enumerate(range(d0, d0 + 8)):
                        plsc.store_scatter(
                            slab, [jrow, jnp.full((L,), d, jnp.int32)], vals[i]
                        )

        slabs = ((rows_a, sem_a), (rows_b, sem_b))

        def row_pair(t, carry):
            for parity, (slab, sem) in enumerate(slabs):
                il = 2 * t + parity

                @pl.when(t > 0)
                def _():
                    pltpu.make_async_copy(
                        slab.at[:, pl.ds(0, D)], out_hbm.at[base], sem
                    ).wait()

                fill_row(il, slab)
                pltpu.async_copy(slab.at[:, pl.ds(0, D)], out_hbm.at[base + il], sem)
            return carry

        lax.fori_loop(0, ROWS_PER_WORKER // 2, row_pair, 0)
        for slab, sem in slabs:
            pltpu.make_async_copy(
                slab.at[:, pl.ds(0, D)], out_hbm.at[base], sem
            ).wait()

    return build_comb, edge_embed


def kernel(residue_index, chain_id, entity_id, sym_id, res_offset_emb, sym_offset_emb):
    build_comb, edge_embed = _make_kernels()
    comb = build_comb(res_offset_emb, sym_offset_emb)
    return edge_embed(residue_index, chain_id, entity_id, sym_id, comb)


# speedup vs baseline: 1.8249x; 1.8249x over previous
import functools

import jax
import jax.numpy as jnp
from jax import lax
from jax.experimental import pallas as pl
from jax.experimental.pallas import tpu as pltpu
from jax.experimental.pallas import tpu_sc as plsc

MAX_RES_OFFSET = 32
MAX_SYM_OFFSET = 2
D = 32
N = 1024
R_ROWS = 2 * MAX_RES_OFFSET + 2
S_ROWS = 2 * MAX_SYM_OFFSET + 2
C_ROWS = R_ROWS * S_ROWS

NUM_CORES = 2
NUM_SUBCORES = 16
NUM_WORKERS = NUM_CORES * NUM_SUBCORES
ROWS_PER_WORKER = N // NUM_WORKERS
L = 16

_SC_PARAMS = dict(
    compiler_params=pltpu.CompilerParams(
        needs_layout_passes=False, use_tc_tiling_on_sc=False
    ),
)


def _worker_id():
    return lax.axis_index("s") * NUM_CORES + lax.axis_index("c")


@functools.cache
def _make_kernels():
    mesh = plsc.VectorSubcoreMesh(core_axis_name="c", subcore_axis_name="s")

    @functools.partial(
        pl.kernel,
        out_type=jax.ShapeDtypeStruct((C_ROWS * D,), jnp.float32),
        mesh=mesh,
        scratch_types=[
            pltpu.VMEM((R_ROWS, D), jnp.float32),
            pltpu.VMEM((S_ROWS, D), jnp.float32),
            pltpu.VMEM((C_ROWS * D,), jnp.float32),
        ],
        **_SC_PARAMS,
    )
    def build_comb(res_hbm, sym_hbm, comb_hbm, res_v, sym_v, comb_v):
        wid = _worker_id()

        @pl.when(wid == 0)
        def _():
            pltpu.sync_copy(res_hbm, res_v)
            pltpu.sync_copy(sym_hbm, sym_v)
            sym_rows = [
                [sym_v[b, pl.ds(h * L, L)] for h in range(D // L)]
                for b in range(S_ROWS)
            ]

            def body(a, carry):
                for h in range(D // L):
                    r = res_v[a, pl.ds(h * L, L)]
                    for b in range(S_ROWS):
                        comb_v[pl.ds((a * S_ROWS + b) * D + h * L, L)] = (
                            r + sym_rows[b][h]
                        )
                return carry

            lax.fori_loop(0, R_ROWS, body, 0)
            pltpu.sync_copy(comb_v, comb_hbm)

    @functools.partial(
        pl.kernel,
        out_type=jax.ShapeDtypeStruct((N, N * D), jnp.float32),
        mesh=mesh,
        scratch_types=[
            pltpu.VMEM((N,), jnp.int32),
            pltpu.VMEM((N,), jnp.int32),
            pltpu.VMEM((N,), jnp.int32),
            pltpu.VMEM((N,), jnp.int32),
            pltpu.VMEM((ROWS_PER_WORKER,), jnp.int32),
            pltpu.VMEM((ROWS_PER_WORKER,), jnp.int32),
            pltpu.VMEM((ROWS_PER_WORKER,), jnp.int32),
            pltpu.VMEM((ROWS_PER_WORKER,), jnp.int32),
            pltpu.VMEM((C_ROWS * D,), jnp.float32),
            pltpu.VMEM((N * D,), jnp.float32),
            pltpu.VMEM((N * D,), jnp.float32),
            pltpu.SemaphoreType.DMA,
            pltpu.SemaphoreType.DMA,
        ],
        **_SC_PARAMS,
    )
    def edge_embed(
        ri_hbm, ch_hbm, en_hbm, sy_hbm, comb_hbm, out_hbm,
        ri_v, ch_v, en_v, sy_v, qri_v, qch_v, qen_v, qsy_v, comb_v,
        rows_a, rows_b, sem_a, sem_b,
    ):
        wid = _worker_id()
        base = wid * ROWS_PER_WORKER

        pltpu.sync_copy(ri_hbm, ri_v)
        pltpu.sync_copy(ch_hbm, ch_v)
        pltpu.sync_copy(en_hbm, en_v)
        pltpu.sync_copy(sy_hbm, sy_v)
        pltpu.sync_copy(ri_hbm.at[pl.ds(base, ROWS_PER_WORKER)], qri_v)
        pltpu.sync_copy(ch_hbm.at[pl.ds(base, ROWS_PER_WORKER)], qch_v)
        pltpu.sync_copy(en_hbm.at[pl.ds(base, ROWS_PER_WORKER)], qen_v)
        pltpu.sync_copy(sy_hbm.at[pl.ds(base, ROWS_PER_WORKER)], qsy_v)
        pltpu.sync_copy(comb_hbm, comb_v)

        iota = lax.iota(jnp.int32, L)
        lane_d = iota * D

        def fill_row(il, slab):
            splat_il = jnp.full((L,), il, jnp.int32)
            q_ri = plsc.load_gather(qri_v, [splat_il])
            q_ch = plsc.load_gather(qch_v, [splat_il])
            q_en = plsc.load_gather(qen_v, [splat_il])
            q_sy = plsc.load_gather(qsy_v, [splat_il])

            @plsc.parallel_loop(0, N // L, 1, unroll=2)
            def grp_body(g):
                j0 = g * L
                rj = ri_v[pl.ds(j0, L)]
                cj = ch_v[pl.ds(j0, L)]
                ej = en_v[pl.ds(j0, L)]
                sj = sy_v[pl.ds(j0, L)]
                rel = jnp.clip(rj - q_ri + MAX_RES_OFFSET, 0, 2 * MAX_RES_OFFSET)
                rel = jnp.where(cj == q_ch, rel, 2 * MAX_RES_OFFSET + 1)
                rs = jnp.clip(sj - q_sy + MAX_SYM_OFFSET, 0, 2 * MAX_SYM_OFFSET)
                rs = jnp.where(ej == q_en, rs, 2 * MAX_SYM_OFFSET + 1)
                src0 = (rel * S_ROWS + rs) * D
                dst0 = lane_d + j0 * D
                for v in range(D):
                    dx = lax.bitwise_xor(iota, v)
                    vals = plsc.load_gather(comb_v, [src0 + dx])
                    plsc.store_scatter(slab, [dst0 + dx], vals)

        slabs = ((rows_a, sem_a), (rows_b, sem_b))

        def row_pair(t, carry):
            for parity, (slab, sem) in enumerate(slabs):
                il = 2 * t + parity

                @pl.when(t > 0)
                def _():
                    pltpu.make_async_copy(slab, out_hbm.at[base], sem).wait()

                fill_row(il, slab)
                pltpu.async_copy(slab, out_hbm.at[base + il], sem)
            return carry

        lax.fori_loop(0, ROWS_PER_WORKER // 2, row_pair, 0)
        for slab, sem in slabs:
            pltpu.make_async_copy(slab, out_hbm.at[base], sem).wait()

    return build_comb, edge_embed


def kernel(residue_index, chain_id, entity_id, sym_id, res_offset_emb, sym_offset_emb):
    build_comb, edge_embed = _make_kernels()
    comb = build_comb(res_offset_emb, sym_offset_emb)
    out = edge_embed(residue_index, chain_id, entity_id, sym_id, comb)
    return out.reshape(N, N, D)
